# Initial kernel scaffold; baseline (speedup 1.0000x reference)
import jax
import jax.numpy as jnp
from jax.experimental import pallas as pl


def kernel(hidden_states, router_logits, W_gate, W_up, W_down):
    raise NotImplementedError("write your pallas kernel here")


# speedup vs baseline: 1.0336x; 1.0336x over previous
import jax
import jax.numpy as jnp
from jax.experimental import pallas as pl
from jax.experimental.pallas import tpu as pltpu

T, H, E, F, TOP_K = 32, 2048, 16, 1408, 2
FB = 128
NF = F // FB


def _routing_scale(logits, e):
    iota = jax.lax.broadcasted_iota(jnp.int32, (T, E), 1)
    m1 = jnp.max(logits, axis=1, keepdims=True)
    idx1 = jnp.min(jnp.where(logits >= m1, iota, E), axis=1, keepdims=True)
    masked = jnp.where(iota == idx1, -jnp.inf, logits)
    m2 = jnp.max(masked, axis=1, keepdims=True)
    idx2 = jnp.min(jnp.where(masked >= m2, iota, E), axis=1, keepdims=True)
    w1 = 1.0 / (1.0 + jnp.exp(m2 - m1))
    w2 = 1.0 - w1
    return jnp.where(idx1 == e, w1, 0.0) + jnp.where(idx2 == e, w2, 0.0)


def _moe_kernel(x_ref, logits_ref, wg_ref, wu_ref, wd_ref, out_ref):
    e = pl.program_id(0)
    f = pl.program_id(1)

    @pl.when((e == 0) & (f == 0))
    def _():
        out_ref[...] = jnp.zeros_like(out_ref)

    x = x_ref[...]
    g = jnp.dot(x, wg_ref[0], preferred_element_type=jnp.float32)
    u = jnp.dot(x, wu_ref[0], preferred_element_type=jnp.float32)
    act = (g * jax.lax.logistic(g)) * u
    partial = jnp.dot(act, wd_ref[0], preferred_element_type=jnp.float32)
    scale = _routing_scale(logits_ref[...], e)
    out_ref[...] += scale * partial


def kernel(hidden_states, router_logits, W_gate, W_up, W_down):
    grid = (E, NF)
    return pl.pallas_call(
        _moe_kernel,
        grid=grid,
        in_specs=[
            pl.BlockSpec((T, H), lambda e, f: (0, 0)),
            pl.BlockSpec((T, E), lambda e, f: (0, 0)),
            pl.BlockSpec((1, H, FB), lambda e, f: (e, 0, f)),
            pl.BlockSpec((1, H, FB), lambda e, f: (e, 0, f)),
            pl.BlockSpec((1, FB, H), lambda e, f: (e, f, 0)),
        ],
        out_specs=pl.BlockSpec((T, H), lambda e, f: (0, 0)),
        out_shape=jax.ShapeDtypeStruct((T, H), jnp.float32),
        compiler_params=pltpu.CompilerParams(
            dimension_semantics=("arbitrary", "arbitrary"),
        ),
    )(hidden_states, router_logits, W_gate, W_up, W_down)


# speedup vs baseline: 1.2166x; 1.1770x over previous
import jax
import jax.numpy as jnp
from jax.experimental import pallas as pl
from jax.experimental.pallas import tpu as pltpu

T, H, E, F, TOP_K = 32, 2048, 16, 1408, 2
HB = 1024
NH = H // HB
NS = NH + 1


def _routing_scale(logits, e):
    iota = jax.lax.broadcasted_iota(jnp.int32, (T, E), 1)
    m1 = jnp.max(logits, axis=1, keepdims=True)
    idx1 = jnp.min(jnp.where(logits >= m1, iota, E), axis=1, keepdims=True)
    masked = jnp.where(iota == idx1, -jnp.inf, logits)
    m2 = jnp.max(masked, axis=1, keepdims=True)
    idx2 = jnp.min(jnp.where(masked >= m2, iota, E), axis=1, keepdims=True)
    w1 = 1.0 / (1.0 + jnp.exp(m2 - m1))
    w2 = 1.0 - w1
    return jnp.where(idx1 == e, w1, 0.0) + jnp.where(idx2 == e, w2, 0.0)


def _moe_kernel(x_ref, logits_ref, wg_ref, wu_ref, wd_ref, out_ref,
                g_ref, u_ref):
    e = pl.program_id(0)
    s = pl.program_id(1)

    @pl.when(s < NH)
    def _phase_a():
        x = x_ref[...]
        g = jnp.dot(x, wg_ref[0], preferred_element_type=jnp.float32)
        u = jnp.dot(x, wu_ref[0], preferred_element_type=jnp.float32)

        @pl.when(s == 0)
        def _():
            g_ref[...] = g
            u_ref[...] = u

        @pl.when(s > 0)
        def _():
            g_ref[...] += g
            u_ref[...] += u

    @pl.when(s == NH)
    def _phase_b():
        g = g_ref[...]
        u = u_ref[...]
        scale = _routing_scale(logits_ref[...], e)
        act = scale * ((g * jax.lax.logistic(g)) * u)
        partial = jnp.dot(act, wd_ref[0], preferred_element_type=jnp.float32)

        @pl.when(e == 0)
        def _():
            out_ref[...] = partial

        @pl.when(e > 0)
        def _():
            out_ref[...] += partial


def kernel(hidden_states, router_logits, W_gate, W_up, W_down):
    return pl.pallas_call(
        _moe_kernel,
        grid=(E, NS),
        in_specs=[
            pl.BlockSpec((T, HB), lambda e, s: (0, jnp.minimum(s, NH - 1))),
            pl.BlockSpec((T, E), lambda e, s: (0, 0)),
            pl.BlockSpec((1, HB, F), lambda e, s: (e, jnp.minimum(s, NH - 1), 0)),
            pl.BlockSpec((1, HB, F), lambda e, s: (e, jnp.minimum(s, NH - 1), 0)),
            pl.BlockSpec((1, F, H), lambda e, s: (e, 0, 0)),
        ],
        out_specs=pl.BlockSpec((T, H), lambda e, s: (0, 0)),
        out_shape=jax.ShapeDtypeStruct((T, H), jnp.float32),
        scratch_shapes=[
            pltpu.VMEM((T, F), jnp.float32),
            pltpu.VMEM((T, F), jnp.float32),
        ],
        compiler_params=pltpu.CompilerParams(
            dimension_semantics=("arbitrary", "arbitrary"),
            vmem_limit_bytes=64 * 1024 * 1024,
        ),
    )(hidden_states, router_logits, W_gate, W_up, W_down)


# speedup vs baseline: 1.2185x; 1.0016x over previous
import jax
import jax.numpy as jnp
from jax.experimental import pallas as pl
from jax.experimental.pallas import tpu as pltpu

T, H, E, F, TOP_K = 32, 2048, 16, 1408, 2
HB = 512
NH = H // HB
HBO = 1024
ND = H // HBO
NS = NH + ND


def _routing_scale(logits, e):
    iota = jax.lax.broadcasted_iota(jnp.int32, (T, E), 1)
    m1 = jnp.max(logits, axis=1, keepdims=True)
    idx1 = jnp.min(jnp.where(logits >= m1, iota, E), axis=1, keepdims=True)
    masked = jnp.where(iota == idx1, -jnp.inf, logits)
    m2 = jnp.max(masked, axis=1, keepdims=True)
    idx2 = jnp.min(jnp.where(masked >= m2, iota, E), axis=1, keepdims=True)
    w1 = 1.0 / (1.0 + jnp.exp(m2 - m1))
    w2 = 1.0 - w1
    return jnp.where(idx1 == e, w1, 0.0) + jnp.where(idx2 == e, w2, 0.0)


def _moe_kernel(x_ref, logits_ref, wg_ref, wu_ref, wd_ref, out_ref,
                g_ref, u_ref, act_ref):
    e = pl.program_id(0)
    s = pl.program_id(1)

    @pl.when(s < NH)
    def _phase_a():
        x = x_ref[...]
        g = jnp.dot(x, wg_ref[0], preferred_element_type=jnp.float32)
        u = jnp.dot(x, wu_ref[0], preferred_element_type=jnp.float32)

        @pl.when(s == 0)
        def _():
            g_ref[...] = g
            u_ref[...] = u

        @pl.when(s > 0)
        def _():
            g_ref[...] += g
            u_ref[...] += u

    @pl.when(s == NH)
    def _make_act():
        g = g_ref[...]
        u = u_ref[...]
        scale = _routing_scale(logits_ref[...], e)
        act_ref[...] = scale * ((g * jax.lax.logistic(g)) * u)

    @pl.when(s >= NH)
    def _phase_b():
        partial = jnp.dot(act_ref[...], wd_ref[0],
                          preferred_element_type=jnp.float32)

        @pl.when(e == 0)
        def _():
            out_ref[...] = partial

        @pl.when(e > 0)
        def _():
            out_ref[...] += partial


def kernel(hidden_states, router_logits, W_gate, W_up, W_down):
    clamp_a = lambda s: jnp.minimum(s, NH - 1)
    clamp_b = lambda s: jnp.clip(s - NH, 0, ND - 1)
    return pl.pallas_call(
        _moe_kernel,
        grid=(E, NS),
        in_specs=[
            pl.BlockSpec((T, HB), lambda e, s: (0, clamp_a(s))),
            pl.BlockSpec((T, E), lambda e, s: (0, 0)),
            pl.BlockSpec((1, HB, F), lambda e, s: (e, clamp_a(s), 0)),
            pl.BlockSpec((1, HB, F), lambda e, s: (e, clamp_a(s), 0)),
            pl.BlockSpec((1, F, HBO), lambda e, s: (e, 0, clamp_b(s))),
        ],
        out_specs=pl.BlockSpec((T, HBO), lambda e, s: (0, clamp_b(s))),
        out_shape=jax.ShapeDtypeStruct((T, H), jnp.float32),
        scratch_shapes=[
            pltpu.VMEM((T, F), jnp.float32),
            pltpu.VMEM((T, F), jnp.float32),
            pltpu.VMEM((T, F), jnp.float32),
        ],
        compiler_params=pltpu.CompilerParams(
            dimension_semantics=("arbitrary", "arbitrary"),
            vmem_limit_bytes=64 * 1024 * 1024,
        ),
    )(hidden_states, router_logits, W_gate, W_up, W_down)
